# baseline (speedup 1.0000x reference)
import functools

import jax
import jax.numpy as jnp
from jax import lax
from jax.experimental import pallas as pl
from jax.experimental.pallas import tpu as pltpu
from jax.experimental.pallas import tpu_sc as plsc

D_MODEL = 1024
VOCAB = 60
LANES = 16
NW = 32
CHUNK = 16
NCHUNK = 32
PER_W = NCHUNK * CHUNK
ROWS = NW * PER_W


def _build():
  mesh = plsc.VectorSubcoreMesh(core_axis_name="c", subcore_axis_name="s")

  @functools.partial(
      pl.kernel,
      mesh=mesh,
      out_type=jax.ShapeDtypeStruct((4, ROWS // 4, 1, D_MODEL), jnp.float32),
      scratch_types=[
          pltpu.VMEM((VOCAB, D_MODEL), jnp.float32),
          pltpu.VMEM((PER_W + LANES,), jnp.int32),
          pltpu.VMEM((CHUNK, D_MODEL), jnp.float32),
          pltpu.VMEM((CHUNK, D_MODEL), jnp.float32),
          pltpu.SemaphoreType.DMA,
          pltpu.SemaphoreType.DMA,
      ],
  )
  def emb(table_hbm, idx_hbm, out_hbm, tab_v, idx_v, buf0, buf1, sem0, sem1):
    wid = lax.axis_index("s") * 2 + lax.axis_index("c")
    pltpu.sync_copy(table_hbm, tab_v)
    pltpu.sync_copy(idx_hbm.at[pl.ds(wid * PER_W, PER_W)],
                    idx_v.at[pl.ds(0, PER_W)])
    row0 = wid * PER_W
    bufs = (buf0, buf1)
    sems = (sem0, sem1)
    cps = [None, None]
    for c in range(NCHUNK):
      b = c % 2
      if cps[b] is not None:
        cps[b].wait()

      def row_body(j, _, c=c, b=b):
        iv = idx_v[pl.ds(c * CHUNK + j, LANES)]
        v = iv[0]
        for k in range(D_MODEL // LANES):
          bufs[b][j, pl.ds(k * LANES, LANES)] = (
              tab_v[v, pl.ds(k * LANES, LANES)])
        return 0

      lax.fori_loop(0, CHUNK, row_body, 0)
      r = row0 + c * CHUNK
      cps[b] = pltpu.async_copy(
          bufs[b], out_hbm.at[r // 4096, pl.ds(r % 4096, CHUNK), 0], sems[b])
    cps[0].wait()
    cps[1].wait()

  return emb


_emb = _build()


def kernel(x_mark, W):
  idx = x_mark[:, :, 1].reshape(ROWS).astype(jnp.int32)
  return _emb(W, idx)


# speedup vs baseline: 3.1883x; 3.1883x over previous
import functools

import jax
import jax.numpy as jnp
from jax import lax
from jax.experimental import pallas as pl
from jax.experimental.pallas import tpu as pltpu
from jax.experimental.pallas import tpu_sc as plsc

D_MODEL = 1024
VOCAB = 60
LANES = 16
NW = 32
GROUP = 16
NGROUP = 32
PER_W = NGROUP * GROUP
ROWS = NW * PER_W


def _build():
  mesh = plsc.VectorSubcoreMesh(core_axis_name="c", subcore_axis_name="s")

  @functools.partial(
      pl.kernel,
      mesh=mesh,
      out_type=jax.ShapeDtypeStruct((4, ROWS // 4, 1, D_MODEL), jnp.float32),
      scratch_types=[
          pltpu.VMEM((VOCAB, D_MODEL), jnp.float32),
          pltpu.VMEM((PER_W + LANES,), jnp.int32),
          pltpu.SemaphoreType.DMA,
      ],
  )
  def emb(table_hbm, idx_hbm, out_hbm, tab_v, idx_v, sem):
    wid = lax.axis_index("s") * 2 + lax.axis_index("c")
    pltpu.sync_copy(table_hbm, tab_v)
    pltpu.sync_copy(idx_hbm.at[pl.ds(wid * PER_W, PER_W)],
                    idx_v.at[pl.ds(0, PER_W)])
    row0 = wid * PER_W
    bidx = row0 // 4096
    t0 = row0 % 4096

    def group_body(g, _):
      cps = []
      for j in range(GROUP):
        iv = idx_v[pl.ds(g * GROUP + j, LANES)]
        v = iv[0]
        cps.append(pltpu.async_copy(
            tab_v.at[pl.ds(v, 1)],
            out_hbm.at[bidx, pl.ds(t0 + g * GROUP + j, 1), 0],
            sem))
      for cp in cps:
        cp.wait()
      return 0

    lax.fori_loop(0, NGROUP, group_body, 0)

  return emb


_emb = _build()


def kernel(x_mark, W):
  idx = x_mark[:, :, 1].reshape(ROWS).astype(jnp.int32)
  return _emb(W, idx)


# speedup vs baseline: 3.3032x; 1.0360x over previous
import functools

import jax
import jax.numpy as jnp
from jax import lax
from jax.experimental import pallas as pl
from jax.experimental.pallas import tpu as pltpu
from jax.experimental.pallas import tpu_sc as plsc

D_MODEL = 1024
VOCAB = 60
LANES = 16
NW = 32
DEPTH = 16
PER_W = 512
ROWS = NW * PER_W


def _build():
  mesh = plsc.VectorSubcoreMesh(core_axis_name="c", subcore_axis_name="s")

  @functools.partial(
      pl.kernel,
      mesh=mesh,
      out_type=jax.ShapeDtypeStruct((4, ROWS // 4, 1, D_MODEL), jnp.float32),
      scratch_types=[
          pltpu.VMEM((VOCAB, D_MODEL), jnp.float32),
          pltpu.VMEM((PER_W + LANES,), jnp.int32),
          pltpu.SemaphoreType.DMA,
      ],
  )
  def emb(table_hbm, idx_hbm, out_hbm, tab_v, idx_v, sem):
    wid = lax.axis_index("s") * 2 + lax.axis_index("c")
    pltpu.sync_copy(table_hbm, tab_v)
    pltpu.sync_copy(idx_hbm.at[pl.ds(wid * PER_W, PER_W)],
                    idx_v.at[pl.ds(0, PER_W)])

    row0 = wid * PER_W
    bidx = row0 // 4096
    t0 = row0 % 4096

    def fire(j):
      iv = idx_v[pl.ds(j, LANES)]
      v = iv[0]
      pltpu.async_copy(
          tab_v.at[pl.ds(v, 1)],
          out_hbm.at[bidx, pl.ds(t0 + j, 1), 0],
          sem)

    def wait_one():
      pltpu.make_async_copy(
          tab_v.at[pl.ds(0, 1)],
          out_hbm.at[bidx, pl.ds(t0, 1), 0],
          sem).wait()

    for j in range(DEPTH):
      fire(j)

    def body(j, _):
      wait_one()
      fire(j + DEPTH)
      return 0

    lax.fori_loop(0, PER_W - DEPTH, body, 0)
    for _ in range(DEPTH):
      wait_one()

  return emb


_emb = _build()


def kernel(x_mark, W):
  idx = x_mark[:, :, 1].reshape(ROWS).astype(jnp.int32)
  return _emb(W, idx)
